# baseline (speedup 1.0000x reference)
import dataclasses
import functools

import jax
import jax.numpy as jnp
from jax import lax
from jax.experimental import pallas as pl
from jax.experimental.pallas import tpu as pltpu
from jax.experimental.pallas import tpu_sc as plsc

N = 100000
D = 128
B = 16384
NC = 2
NS = 16
L = 16

R = 4096
NCHUNK = 25
TAIL = N - (NCHUNK - 1) * R
SLOTS = (NCHUNK + NC - 1) // NC
ROWS_PER_TILE = R // NS
TAIL_RPT = 128
SCAN_PER_TILE = B // NS
NVEC = SCAN_PER_TILE // L
KB = 64
KBSH = KB.bit_length() - 1
MAXM = SCAN_PER_TILE + KB
NBROWS = MAXM // KB


def _sc_index_add(dst, src, idx):
  mesh = plsc.VectorSubcoreMesh(
      core_axis_name="c", subcore_axis_name="s",
      num_cores=NC, num_subcores=NS)
  cp = pltpu.CompilerParams()
  if "needs_layout_passes" in pltpu.CompilerParams.__dataclass_fields__:
    cp = dataclasses.replace(cp, needs_layout_passes=False)

  lists_t = [
      pltpu.VMEM((MAXM,), jnp.int32),
      pltpu.VMEM((NBROWS, KB), jnp.int32),
      pltpu.VMEM((NBROWS, KB), jnp.int32),
      pltpu.VMEM((NBROWS, KB), jnp.int32),
      pltpu.VMEM((KB, D), jnp.float32),
      pltpu.SemaphoreType.DMA,
  ]

  @functools.partial(
      pl.kernel,
      out_type=jax.ShapeDtypeStruct((N, D), jnp.float32),
      mesh=mesh,
      compiler_params=cp,
      scratch_types=(
          [pltpu.VMEM_SHARED((R + L, D), jnp.float32)] * 2
          + [pltpu.VMEM((SCAN_PER_TILE,), jnp.int32)]
          + lists_t + lists_t
          + [pltpu.SMEM((2,), jnp.int32)]
          + [pltpu.SemaphoreType.DMA] * 4
      ),
  )
  def run(dst_hbm, src_hbm, idx_hbm, out_hbm,
          acc0, acc1, idxbuf,
          posA, lidxA, sA, gA, stgA, gsemA,
          posB, lidxB, sB, gB, stgB, gsemB,
          msc, lsem0, lsem1, csem0, csem1):
    core = lax.axis_index("c")
    sub = lax.axis_index("s")
    lanes = lax.iota(jnp.int32, L)
    ones = lanes >= 0

    def tile_slices(k_local):
      if k_local == SLOTS - 1:
        return ((sub < 13, sub * TAIL_RPT, TAIL_RPT),
                (sub == 13, 13 * TAIL_RPT, TAIL - 13 * TAIL_RPT))
      return ((None, sub * ROWS_PER_TILE, ROWS_PER_TILE),)

    def _dense(k_local, acc, sem, fn):
      base = (k_local * NC + core) * R
      for pred, off, nrows in tile_slices(k_local):
        def emit(off=off, nrows=nrows):
          fn(dst_hbm.at[pl.ds(base + off, nrows)],
             out_hbm.at[pl.ds(base + off, nrows)],
             acc.at[pl.ds(off, nrows)], sem)
        if pred is None:
          emit()
        else:
          pl.when(pred)(emit)

    def load_issue(k_local, acc, sem):
      _dense(k_local, acc, sem,
             lambda d, o, a, s: pltpu.async_copy(d, a, s))

    def load_wait(k_local, acc, sem):
      _dense(k_local, acc, sem,
             lambda d, o, a, s: pltpu.make_async_copy(d, a, s).wait())

    def copy_issue(k_local, acc, sem):
      _dense(k_local, acc, sem,
             lambda d, o, a, s: pltpu.async_copy(d, o, s))

    def copy_wait(k_local, acc, sem):
      _dense(k_local, acc, sem,
             lambda d, o, a, s: pltpu.make_async_copy(d, o, s).wait())

    def scan_chunk(k_local, pos, lidx, s2d, g2d, stg, gsem, par):
      base = (k_local * NC + core) * R
      size = TAIL if k_local == SLOTS - 1 else R

      def scan_body(v, m_vec):
        vec = idxbuf[pl.ds(v * L, L)]
        rel = vec - base
        mask = rel.astype(jnp.uint32) < jnp.uint32(size)
        mi = mask.astype(jnp.int32)
        off = m_vec + plsc.cumsum(mi) - mi
        p = lanes + (sub * SCAN_PER_TILE + v * L)
        rowsel = (off >> KBSH, off & (KB - 1))
        plsc.store_scatter(pos, [off], p, mask=mask)
        plsc.store_scatter(lidx, list(rowsel), rel, mask=mask)
        plsc.store_scatter(s2d, list(rowsel), rel, mask=mask)
        plsc.store_scatter(g2d, list(rowsel), vec, mask=mask)
        return m_vec + plsc.all_reduce_population_count(mask)

      m_vec = lax.fori_loop(0, NVEC, scan_body, jnp.zeros((L,), jnp.int32),
                            unroll=4)
      m = jnp.max(m_vec)
      msc[par] = m

      @pl.loop(0, KB // L)
      def _pad(j):
        off_pad = m + lanes + j * L
        rowsel = (off_pad >> KBSH, off_pad & (KB - 1))
        plsc.store_scatter(pos, [off_pad], lanes + j * L, mask=ones)
        plsc.store_scatter(lidx, list(rowsel), lanes + R, mask=ones)
        plsc.store_scatter(s2d, list(rowsel), lanes * 0, mask=ones)
        plsc.store_scatter(g2d, list(rowsel), lanes * 0 + base, mask=ones)

      pltpu.async_copy(src_hbm.at[pos.at[pl.ds(0, KB)]], stg, gsem)

    def add_phase(acc, pos, lidx, s2d, g2d, stg, gsem, par):
      pltpu.make_async_copy(src_hbm.at[pos.at[pl.ds(0, KB)]], stg,
                            gsem).wait()
      pltpu.sync_copy(stg, acc.at[lidx.at[0]], add=True)
      nb = (msc[par] + (KB - 1)) >> KBSH

      def batch_body(b, carry):
        pltpu.sync_copy(src_hbm.at[pos.at[pl.ds(b * KB, KB)]], stg)
        pltpu.sync_copy(stg, acc.at[lidx.at[b]], add=True)
        return carry

      lax.fori_loop(1, nb, batch_body, jnp.int32(0))

    def scatter_out(acc, pos, lidx, s2d, g2d, stg, gsem, par):
      nb = (msc[par] + (KB - 1)) >> KBSH

      def batch_body(b, carry):
        pltpu.sync_copy(acc.at[s2d.at[b]], stg)
        pltpu.sync_copy(stg, out_hbm.at[g2d.at[b]])
        return carry

      pltpu.sync_copy(acc.at[s2d.at[0]], stg)
      pltpu.sync_copy(stg, out_hbm.at[g2d.at[0]])
      lax.fori_loop(1, nb, batch_body, jnp.int32(0))

    sets = ((posA, lidxA, sA, gA, stgA, gsemA),
            (posB, lidxB, sB, gB, stgB, gsemB))
    bufs = ((acc0, lsem0, csem0), (acc1, lsem1, csem1))
    my_kpc = jnp.int32(SLOTS - 1) + (core == 0).astype(jnp.int32)

    for kk in range(2):
      load_issue(kk, bufs[kk][0], bufs[kk][1])
      copy_issue(kk, bufs[kk][0], bufs[kk][2])

    pltpu.sync_copy(idx_hbm.at[pl.ds(sub * SCAN_PER_TILE, SCAN_PER_TILE)],
                    idxbuf)
    scan_chunk(0, *sets[0], 0)

    for k in range(SLOTS):
      acc, lsem, csem = bufs[k % 2]
      par = k % 2

      def one_slot(k=k, acc=acc, lsem=lsem, csem=csem, par=par):
        if k + 1 < SLOTS:
          with jax.named_scope("ph_prefetch"):
            @pl.when(jnp.int32(k + 1) < my_kpc)
            def _p():
              scan_chunk(k + 1, *sets[1 - par], 1 - par)

        with jax.named_scope("ph_ldwait"):
          load_wait(k, acc, lsem)
          plsc.subcore_barrier()
        with jax.named_scope("ph_add"):
          add_phase(acc, *sets[par], par)
        with jax.named_scope("ph_cpwait"):
          copy_wait(k, acc, csem)
        with jax.named_scope("ph_bar"):
          plsc.subcore_barrier()
        with jax.named_scope("ph_scatter"):
          scatter_out(acc, *sets[par], par)

        if k + 2 < SLOTS:
          with jax.named_scope("ph_reissue"):
            plsc.subcore_barrier()

            @pl.when(jnp.int32(k + 2) < my_kpc)
            def _r():
              load_issue(k + 2, acc, lsem)
              copy_issue(k + 2, acc, csem)

      if k < SLOTS - 1:
        one_slot()
      else:
        @pl.when(jnp.int32(k) < my_kpc)
        def _last():
          one_slot()

  return run(dst, src, idx)


def kernel(dst_tensor, src_tensor, index_tensor):
  return _sc_index_add(dst_tensor, src_tensor,
                       index_tensor.astype(jnp.int32))


# speedup vs baseline: 19.1379x; 19.1379x over previous
import dataclasses
import functools

import jax
import jax.numpy as jnp
from jax import lax
from jax.experimental import pallas as pl
from jax.experimental.pallas import tpu as pltpu
from jax.experimental.pallas import tpu_sc as plsc

N = 100000
D = 128
B = 16384
NC = 2
NS = 16
L = 16

NCHUNK = 20
R = N // NCHUNK
KPC = NCHUNK // NC
DENSE_TILES = 5
ROWS_PER_TILE = R // DENSE_TILES
SCAN_PER_TILE = B // NS
NVEC = SCAN_PER_TILE // L
KB = 64
KBSH = KB.bit_length() - 1
MAXM = SCAN_PER_TILE + KB
NBROWS = MAXM // KB


def _sc_index_add(dst, src, idx):
  mesh = plsc.VectorSubcoreMesh(
      core_axis_name="c", subcore_axis_name="s",
      num_cores=NC, num_subcores=NS)
  cp = pltpu.CompilerParams()
  if "needs_layout_passes" in pltpu.CompilerParams.__dataclass_fields__:
    cp = dataclasses.replace(cp, needs_layout_passes=False)

  @functools.partial(
      pl.kernel,
      out_type=jax.ShapeDtypeStruct((N, D), jnp.float32),
      mesh=mesh,
      compiler_params=cp,
      scratch_types=[
          pltpu.VMEM_SHARED((R + L, D), jnp.float32),
          pltpu.VMEM_SHARED((R + L, D), jnp.float32),
          pltpu.VMEM((SCAN_PER_TILE,), jnp.int32),
          pltpu.VMEM((MAXM,), jnp.int32),
          pltpu.VMEM((MAXM,), jnp.int32),
          pltpu.VMEM((NBROWS, KB), jnp.int32),
          pltpu.VMEM((NBROWS, KB), jnp.int32),
          pltpu.VMEM((KB, D), jnp.float32),
          pltpu.VMEM((KB, D), jnp.float32),
          pltpu.SMEM((2,), jnp.int32),
          pltpu.SemaphoreType.DMA,
          pltpu.SemaphoreType.DMA,
          pltpu.SemaphoreType.DMA,
          pltpu.SemaphoreType.DMA,
          pltpu.SemaphoreType.DMA,
          pltpu.SemaphoreType.DMA,
      ],
  )
  def run(dst_hbm, src_hbm, idx_hbm, out_hbm,
          acc0, acc1, idxbuf, posA, posB, lidxA, lidxB, stgA, stgB, msc,
          lsem0, lsem1, ssem0, ssem1, gsemA, gsemB):
    core = lax.axis_index("c")
    sub = lax.axis_index("s")
    lanes = lax.iota(jnp.int32, L)
    ones = lanes >= 0

    def hbm_slc(k_local):
      base = (k_local * NC + core) * R
      return dst_hbm.at[pl.ds(base + sub * ROWS_PER_TILE, ROWS_PER_TILE)]

    def out_slc(k_local):
      base = (k_local * NC + core) * R
      return out_hbm.at[pl.ds(base + sub * ROWS_PER_TILE, ROWS_PER_TILE)]

    def acc_slc(acc):
      return acc.at[pl.ds(sub * ROWS_PER_TILE, ROWS_PER_TILE)]

    def load_issue(k_local, acc, sem):
      pltpu.async_copy(hbm_slc(k_local), acc_slc(acc), sem)

    def load_wait(k_local, acc, sem):
      pltpu.make_async_copy(hbm_slc(k_local), acc_slc(acc), sem).wait()

    def store_issue(k_local, acc, sem):
      pltpu.async_copy(acc_slc(acc), out_slc(k_local), sem)

    def store_wait(k_local, acc, sem):
      pltpu.make_async_copy(acc_slc(acc), out_slc(k_local), sem).wait()

    def scan_chunk(k_local, pos, lidx, stg, gsem, par):
      base = (k_local * NC + core) * R

      def scan_body(v, m_vec):
        vec = idxbuf[pl.ds(v * L, L)]
        rel = vec - base
        mask = rel.astype(jnp.uint32) < jnp.uint32(R)
        mi = mask.astype(jnp.int32)
        off = m_vec + plsc.cumsum(mi) - mi
        p = lanes + (sub * SCAN_PER_TILE + v * L)
        plsc.store_scatter(pos, [off], p, mask=mask)
        plsc.store_scatter(lidx, [off >> KBSH, off & (KB - 1)], rel,
                           mask=mask)
        return m_vec + plsc.all_reduce_population_count(mask)

      m_vec = lax.fori_loop(0, NVEC, scan_body, jnp.zeros((L,), jnp.int32),
                            unroll=4)
      m = jnp.max(m_vec)
      msc[par] = m

      @pl.loop(0, KB // L)
      def _pad(j):
        off_pad = m + lanes + j * L
        plsc.store_scatter(pos, [off_pad], lanes + j * L, mask=ones)
        plsc.store_scatter(lidx, [off_pad >> KBSH, off_pad & (KB - 1)],
                           lanes + R, mask=ones)

      pltpu.async_copy(src_hbm.at[pos.at[pl.ds(0, KB)]], stg, gsem)

    def add_phase(acc, pos, lidx, stg, gsem, par):
      pltpu.make_async_copy(src_hbm.at[pos.at[pl.ds(0, KB)]], stg,
                            gsem).wait()
      pltpu.sync_copy(stg, acc.at[lidx.at[0]], add=True)
      nb = (msc[par] + (KB - 1)) >> KBSH

      def batch_body(b, carry):
        pltpu.sync_copy(src_hbm.at[pos.at[pl.ds(b * KB, KB)]], stg)
        pltpu.sync_copy(stg, acc.at[lidx.at[b]], add=True)
        return carry

      lax.fori_loop(1, nb, batch_body, jnp.int32(0))

    sets = ((posA, lidxA, stgA, gsemA), (posB, lidxB, stgB, gsemB))

    def slot(k, acc, lsem, ssem, par):
      @pl.when(jnp.int32(k + 1) < KPC)
      def _prefetch():
        scan_chunk(k + 1, *sets[1 - par], 1 - par)

      @pl.when(sub < DENSE_TILES)
      def _w():
        load_wait(k, acc, lsem)
      plsc.subcore_barrier()
      add_phase(acc, *sets[par], par)
      plsc.subcore_barrier()

      @pl.when(sub < DENSE_TILES)
      def _s():
        store_issue(k, acc, ssem)

    @pl.when(sub < DENSE_TILES)
    def _prime():
      load_issue(0, acc0, lsem0)
      load_issue(1, acc1, lsem1)

    pltpu.sync_copy(idx_hbm.at[pl.ds(sub * SCAN_PER_TILE, SCAN_PER_TILE)],
                    idxbuf)
    scan_chunk(0, *sets[0], 0)

    @pl.loop(0, KPC // 2)
    def _pair(j):
      k0 = 2 * j
      k1 = 2 * j + 1
      slot(k0, acc0, lsem0, ssem0, 0)
      slot(k1, acc1, lsem1, ssem1, 1)

      @pl.when(jnp.logical_and(sub < DENSE_TILES, j < KPC // 2 - 1))
      def _reissue():
        store_wait(k0, acc0, ssem0)
        load_issue(k0 + 2, acc0, lsem0)
        store_wait(k1, acc1, ssem1)
        load_issue(k1 + 2, acc1, lsem1)

    @pl.when(sub < DENSE_TILES)
    def _drain():
      store_wait(KPC - 2, acc0, ssem0)
      store_wait(KPC - 1, acc1, ssem1)

  return run(dst, src, idx)


def kernel(dst_tensor, src_tensor, index_tensor):
  return _sc_index_add(dst_tensor, src_tensor,
                       index_tensor.astype(jnp.int32))
